# baseline (speedup 1.0000x reference)
import functools

import jax
import jax.numpy as jnp
from jax import lax
from jax.experimental import pallas as pl
from jax.experimental.pallas import tpu as pltpu
from jax.experimental.pallas import tpu_sc as plsc

N = 50000
E = 1_600_000
NC = 2
NS = 16
NW = NC * NS
BMAX = 3200

SPLITS = (200_192, 699_904, 699_904)


def _chunk_schedule(epw):
    full = epw // BMAX
    chunks = [(k * BMAX, BMAX) for k in range(full)]
    if epw - full * BMAX:
        chunks.append((full * BMAX, epw - full * BMAX))
    return chunks


def _make_body(epw, part_base):
    chunks = _chunk_schedule(epw)
    nch = len(chunks)

    def _body(rx_hbm, ry_hbm, rz_hbm, ii_hbm, ij_hbm,
              outx_hbm, outy_hbm, outz_hbm,
              tx, ty, tz,
              idx0, idx1,
              gx0, gy0, gz0, gx1, gy1, gz1,
              sin0, sin1, sg0, sg1, sout0, sout1):
        c = lax.axis_index("c")
        s = lax.axis_index("s")
        wid = s * NC + c

        @pl.when(s == 0)
        def _():
            pltpu.sync_copy(rx_hbm, tx)

        @pl.when(s == 1)
        def _():
            pltpu.sync_copy(ry_hbm, ty)

        @pl.when(s == 2)
        def _():
            pltpu.sync_copy(rz_hbm, tz)

        plsc.subcore_barrier()

        lbase = wid * epw
        ebase = part_base + lbase
        idxs = [(idx0, sin0), (idx1, sin1)]
        gs = [(gx0, gy0, gz0, sg0), (gx1, gy1, gz1, sg1)]
        souts = [sout0, sout1]

        def in_descs(ch):
            off, sz = chunks[ch]
            idx_v, sem = idxs[ch % 2]
            sl = pl.ds(ebase + off, sz)
            return (pltpu.make_async_copy(ii_hbm.at[sl], idx_v.at[pl.ds(0, sz)], sem),
                    pltpu.make_async_copy(ij_hbm.at[sl], idx_v.at[pl.ds(BMAX, sz)], sem))

        def g_descs(ch):
            _, sz = chunks[ch]
            idx_v, _ = idxs[ch % 2]
            gx, gy, gz, sem = gs[ch % 2]
            return (
                pltpu.make_async_copy(tx.at[idx_v.at[pl.ds(0, sz)]], gx.at[pl.ds(0, sz)], sem),
                pltpu.make_async_copy(ty.at[idx_v.at[pl.ds(0, sz)]], gy.at[pl.ds(0, sz)], sem),
                pltpu.make_async_copy(tz.at[idx_v.at[pl.ds(0, sz)]], gz.at[pl.ds(0, sz)], sem),
                pltpu.make_async_copy(tx.at[idx_v.at[pl.ds(BMAX, sz)]], gx.at[pl.ds(BMAX, sz)], sem),
                pltpu.make_async_copy(ty.at[idx_v.at[pl.ds(BMAX, sz)]], gy.at[pl.ds(BMAX, sz)], sem),
                pltpu.make_async_copy(tz.at[idx_v.at[pl.ds(BMAX, sz)]], gz.at[pl.ds(BMAX, sz)], sem),
            )

        def out_descs(ch):
            off, sz = chunks[ch]
            gx, gy, gz, _ = gs[ch % 2]
            sem = souts[ch % 2]
            sl = pl.ds(lbase + off, sz)
            return (pltpu.make_async_copy(gx.at[pl.ds(0, sz)], outx_hbm.at[sl], sem),
                    pltpu.make_async_copy(gy.at[pl.ds(0, sz)], outy_hbm.at[sl], sem),
                    pltpu.make_async_copy(gz.at[pl.ds(0, sz)], outz_hbm.at[sl], sem))

        def compute(ch):
            _, sz = chunks[ch]
            gx, gy, gz, _ = gs[ch % 2]

            def ew(v, carry):
                vi = pl.ds(v * 16, 16)
                vj = pl.ds(BMAX + v * 16, 16)
                gx[vi] = gx[vj] - gx[vi]
                gy[vi] = gy[vj] - gy[vi]
                gz[vi] = gz[vj] - gz[vi]
                return carry

            lax.fori_loop(0, sz // 16, ew, 0, unroll=5)

        for d in in_descs(0):
            d.start()
        for d in in_descs(0):
            d.wait()
        for d in g_descs(0):
            d.start()
        for d in in_descs(1):
            d.start()

        for ch in range(nch):
            if ch + 1 < nch:
                for d in in_descs(ch + 1):
                    d.wait()
                if ch >= 1:
                    for d in out_descs(ch - 1):
                        d.wait()
                for d in g_descs(ch + 1):
                    d.start()
            for d in g_descs(ch):
                d.wait()
            if ch + 2 < nch:
                for d in in_descs(ch + 2):
                    d.start()
            compute(ch)
            for d in out_descs(ch):
                d.start()

        for d in out_descs(nch - 2):
            d.wait()
        for d in out_descs(nch - 1):
            d.wait()

    return _body


def _make_kernel(esub, part_base):
    return functools.partial(
        pl.kernel,
        mesh=plsc.VectorSubcoreMesh(core_axis_name="c", subcore_axis_name="s"),
        out_type=(
            jax.ShapeDtypeStruct((esub,), jnp.float32),
            jax.ShapeDtypeStruct((esub,), jnp.float32),
            jax.ShapeDtypeStruct((esub,), jnp.float32),
        ),
        compiler_params=pltpu.CompilerParams(
            needs_layout_passes=False, use_tc_tiling_on_sc=False),
        scratch_types=[
            pltpu.VMEM_SHARED((N,), jnp.float32),
            pltpu.VMEM_SHARED((N,), jnp.float32),
            pltpu.VMEM_SHARED((N,), jnp.float32),
            pltpu.VMEM((2 * BMAX,), jnp.int32),
            pltpu.VMEM((2 * BMAX,), jnp.int32),
            pltpu.VMEM((2 * BMAX,), jnp.float32),
            pltpu.VMEM((2 * BMAX,), jnp.float32),
            pltpu.VMEM((2 * BMAX,), jnp.float32),
            pltpu.VMEM((2 * BMAX,), jnp.float32),
            pltpu.VMEM((2 * BMAX,), jnp.float32),
            pltpu.VMEM((2 * BMAX,), jnp.float32),
            pltpu.SemaphoreType.DMA,
            pltpu.SemaphoreType.DMA,
            pltpu.SemaphoreType.DMA,
            pltpu.SemaphoreType.DMA,
            pltpu.SemaphoreType.DMA,
            pltpu.SemaphoreType.DMA,
        ],
    )(lambda *refs: _make_body(esub // NW, part_base)(*refs))


_part_bases = [sum(SPLITS[:k]) for k in range(len(SPLITS))]
_sc_parts = [_make_kernel(esub, base)
             for esub, base in zip(SPLITS, _part_bases)]


@jax.jit
def kernel(R, offsets, idx_i, idx_j):
    rx, ry, rz = R[:, 0], R[:, 1], R[:, 2]
    ii = idx_i.astype(jnp.int32)
    ij = idx_j.astype(jnp.int32)
    parts = []
    base = 0
    for esub, sc_part in zip(SPLITS, _sc_parts):
        dx, dy, dz = sc_part(rx, ry, rz, ii, ij)
        parts.append(jnp.stack([dx, dy, dz], axis=-1)
                     + offsets[base:base + esub])
        base += esub
    return jnp.concatenate(parts, axis=0)


# speedup vs baseline: 1.3114x; 1.3114x over previous
import functools

import jax
import jax.numpy as jnp
from jax import lax
from jax.experimental import pallas as pl
from jax.experimental.pallas import tpu as pltpu
from jax.experimental.pallas import tpu_sc as plsc

N = 50000
E = 1_600_000
NC = 2
NS = 16
NW = NC * NS
BMAX = 4000

SPLITS = (800_768, 799_232)


def _chunk_schedule(epw):
    full = epw // BMAX
    chunks = [(k * BMAX, BMAX) for k in range(full)]
    if epw - full * BMAX:
        chunks.append((full * BMAX, epw - full * BMAX))
    return chunks


def _make_body(epw, part_base):
    chunks = _chunk_schedule(epw)
    nch = len(chunks)

    def _body(rx_hbm, ry_hbm, rz_hbm, ii_hbm, ij_hbm,
              outx_hbm, outy_hbm, outz_hbm,
              tx, ty, tz,
              idx0, idx1,
              gx0, gy0, gz0, gx1, gy1, gz1,
              sin0, sin1, sg0, sg1, sout0, sout1):
        c = lax.axis_index("c")
        s = lax.axis_index("s")
        wid = s * NC + c

        @pl.when(s == 0)
        def _():
            pltpu.sync_copy(rx_hbm, tx)

        @pl.when(s == 1)
        def _():
            pltpu.sync_copy(ry_hbm, ty)

        @pl.when(s == 2)
        def _():
            pltpu.sync_copy(rz_hbm, tz)

        plsc.subcore_barrier()

        lbase = wid * epw
        ebase = part_base + lbase
        idxs = [(idx0, sin0), (idx1, sin1)]
        gs = [(gx0, gy0, gz0, sg0), (gx1, gy1, gz1, sg1)]
        souts = [sout0, sout1]

        def in_descs(ch):
            off, sz = chunks[ch]
            idx_v, sem = idxs[ch % 2]
            sl = pl.ds(ebase + off, sz)
            return (pltpu.make_async_copy(ii_hbm.at[sl], idx_v.at[pl.ds(0, sz)], sem),
                    pltpu.make_async_copy(ij_hbm.at[sl], idx_v.at[pl.ds(BMAX, sz)], sem))

        def g_descs(ch):
            _, sz = chunks[ch]
            idx_v, _ = idxs[ch % 2]
            gx, gy, gz, sem = gs[ch % 2]
            return (
                pltpu.make_async_copy(tx.at[idx_v.at[pl.ds(0, sz)]], gx.at[pl.ds(0, sz)], sem),
                pltpu.make_async_copy(ty.at[idx_v.at[pl.ds(0, sz)]], gy.at[pl.ds(0, sz)], sem),
                pltpu.make_async_copy(tz.at[idx_v.at[pl.ds(0, sz)]], gz.at[pl.ds(0, sz)], sem),
                pltpu.make_async_copy(tx.at[idx_v.at[pl.ds(BMAX, sz)]], gx.at[pl.ds(BMAX, sz)], sem),
                pltpu.make_async_copy(ty.at[idx_v.at[pl.ds(BMAX, sz)]], gy.at[pl.ds(BMAX, sz)], sem),
                pltpu.make_async_copy(tz.at[idx_v.at[pl.ds(BMAX, sz)]], gz.at[pl.ds(BMAX, sz)], sem),
            )

        def out_descs(ch):
            off, sz = chunks[ch]
            gx, gy, gz, _ = gs[ch % 2]
            sem = souts[ch % 2]
            sl = pl.ds(lbase + off, sz)
            return (pltpu.make_async_copy(gx.at[pl.ds(0, sz)], outx_hbm.at[sl], sem),
                    pltpu.make_async_copy(gy.at[pl.ds(0, sz)], outy_hbm.at[sl], sem),
                    pltpu.make_async_copy(gz.at[pl.ds(0, sz)], outz_hbm.at[sl], sem))

        def compute(ch):
            _, sz = chunks[ch]
            gx, gy, gz, _ = gs[ch % 2]

            def ew(v, carry):
                vi = pl.ds(v * 16, 16)
                vj = pl.ds(BMAX + v * 16, 16)
                gx[vi] = gx[vj] - gx[vi]
                gy[vi] = gy[vj] - gy[vi]
                gz[vi] = gz[vj] - gz[vi]
                return carry

            lax.fori_loop(0, sz // 16, ew, 0, unroll=5)

        for d in in_descs(0):
            d.start()
        for d in in_descs(0):
            d.wait()
        for d in g_descs(0):
            d.start()
        for d in in_descs(1):
            d.start()

        for ch in range(nch):
            if ch + 1 < nch:
                for d in in_descs(ch + 1):
                    d.wait()
                if ch >= 1:
                    for d in out_descs(ch - 1):
                        d.wait()
                for d in g_descs(ch + 1):
                    d.start()
            for d in g_descs(ch):
                d.wait()
            if ch + 2 < nch:
                for d in in_descs(ch + 2):
                    d.start()
            compute(ch)
            for d in out_descs(ch):
                d.start()

        for d in out_descs(nch - 2):
            d.wait()
        for d in out_descs(nch - 1):
            d.wait()

    return _body


def _make_kernel(esub, part_base):
    return functools.partial(
        pl.kernel,
        mesh=plsc.VectorSubcoreMesh(core_axis_name="c", subcore_axis_name="s"),
        out_type=(
            jax.ShapeDtypeStruct((esub,), jnp.float32),
            jax.ShapeDtypeStruct((esub,), jnp.float32),
            jax.ShapeDtypeStruct((esub,), jnp.float32),
        ),
        compiler_params=pltpu.CompilerParams(
            needs_layout_passes=False, use_tc_tiling_on_sc=False),
        scratch_types=[
            pltpu.VMEM_SHARED((N,), jnp.float32),
            pltpu.VMEM_SHARED((N,), jnp.float32),
            pltpu.VMEM_SHARED((N,), jnp.float32),
            pltpu.VMEM((2 * BMAX,), jnp.int32),
            pltpu.VMEM((2 * BMAX,), jnp.int32),
            pltpu.VMEM((2 * BMAX,), jnp.float32),
            pltpu.VMEM((2 * BMAX,), jnp.float32),
            pltpu.VMEM((2 * BMAX,), jnp.float32),
            pltpu.VMEM((2 * BMAX,), jnp.float32),
            pltpu.VMEM((2 * BMAX,), jnp.float32),
            pltpu.VMEM((2 * BMAX,), jnp.float32),
            pltpu.SemaphoreType.DMA,
            pltpu.SemaphoreType.DMA,
            pltpu.SemaphoreType.DMA,
            pltpu.SemaphoreType.DMA,
            pltpu.SemaphoreType.DMA,
            pltpu.SemaphoreType.DMA,
        ],
    )(lambda *refs: _make_body(esub // NW, part_base)(*refs))


_part_bases = [sum(SPLITS[:k]) for k in range(len(SPLITS))]
_sc_parts = [_make_kernel(esub, base)
             for esub, base in zip(SPLITS, _part_bases)]


@jax.jit
def kernel(R, offsets, idx_i, idx_j):
    rx, ry, rz = R[:, 0], R[:, 1], R[:, 2]
    ii = idx_i.astype(jnp.int32)
    ij = idx_j.astype(jnp.int32)
    parts = []
    base = 0
    for esub, sc_part in zip(SPLITS, _sc_parts):
        dx, dy, dz = sc_part(rx, ry, rz, ii, ij)
        parts.append(jnp.stack([dx, dy, dz], axis=-1)
                     + offsets[base:base + esub])
        base += esub
    return jnp.concatenate(parts, axis=0)


# speedup vs baseline: 1.3196x; 1.0062x over previous
import functools

import jax
import jax.numpy as jnp
from jax import lax
from jax.experimental import pallas as pl
from jax.experimental.pallas import tpu as pltpu
from jax.experimental.pallas import tpu_sc as plsc

N = 50000
E = 1_600_000
NC = 2
NS = 16
NW = NC * NS
BMAX = 3200

SPLITS = (800_768, 799_232)


def _chunk_schedule(epw):
    full = epw // BMAX
    chunks = [(k * BMAX, BMAX) for k in range(full)]
    if epw - full * BMAX:
        chunks.append((full * BMAX, epw - full * BMAX))
    return chunks


def _make_body(epw, part_base):
    chunks = _chunk_schedule(epw)
    nch = len(chunks)

    def _body(rx_hbm, ry_hbm, rz_hbm, ii_hbm, ij_hbm,
              outx_hbm, outy_hbm, outz_hbm,
              tx, ty, tz,
              idx0, idx1,
              gx0, gy0, gz0, gx1, gy1, gz1,
              sin0, sin1, sg0, sg1, sout0, sout1):
        c = lax.axis_index("c")
        s = lax.axis_index("s")
        wid = s * NC + c

        @pl.when(s == 0)
        def _():
            pltpu.sync_copy(rx_hbm, tx)

        @pl.when(s == 1)
        def _():
            pltpu.sync_copy(ry_hbm, ty)

        @pl.when(s == 2)
        def _():
            pltpu.sync_copy(rz_hbm, tz)

        plsc.subcore_barrier()

        lbase = wid * epw
        ebase = part_base + lbase
        idxs = [(idx0, sin0), (idx1, sin1)]
        gs = [(gx0, gy0, gz0, sg0), (gx1, gy1, gz1, sg1)]
        souts = [sout0, sout1]

        def in_descs(ch):
            off, sz = chunks[ch]
            idx_v, sem = idxs[ch % 2]
            sl = pl.ds(ebase + off, sz)
            return (pltpu.make_async_copy(ii_hbm.at[sl], idx_v.at[pl.ds(0, sz)], sem),
                    pltpu.make_async_copy(ij_hbm.at[sl], idx_v.at[pl.ds(BMAX, sz)], sem))

        def g_descs(ch):
            _, sz = chunks[ch]
            idx_v, _ = idxs[ch % 2]
            gx, gy, gz, sem = gs[ch % 2]
            return (
                pltpu.make_async_copy(tx.at[idx_v.at[pl.ds(0, sz)]], gx.at[pl.ds(0, sz)], sem),
                pltpu.make_async_copy(ty.at[idx_v.at[pl.ds(0, sz)]], gy.at[pl.ds(0, sz)], sem),
                pltpu.make_async_copy(tz.at[idx_v.at[pl.ds(0, sz)]], gz.at[pl.ds(0, sz)], sem),
                pltpu.make_async_copy(tx.at[idx_v.at[pl.ds(BMAX, sz)]], gx.at[pl.ds(BMAX, sz)], sem),
                pltpu.make_async_copy(ty.at[idx_v.at[pl.ds(BMAX, sz)]], gy.at[pl.ds(BMAX, sz)], sem),
                pltpu.make_async_copy(tz.at[idx_v.at[pl.ds(BMAX, sz)]], gz.at[pl.ds(BMAX, sz)], sem),
            )

        def out_descs(ch):
            off, sz = chunks[ch]
            gx, gy, gz, _ = gs[ch % 2]
            sem = souts[ch % 2]
            sl = pl.ds(lbase + off, sz)
            return (pltpu.make_async_copy(gx.at[pl.ds(0, sz)], outx_hbm.at[sl], sem),
                    pltpu.make_async_copy(gy.at[pl.ds(0, sz)], outy_hbm.at[sl], sem),
                    pltpu.make_async_copy(gz.at[pl.ds(0, sz)], outz_hbm.at[sl], sem))

        def compute(ch):
            _, sz = chunks[ch]
            gx, gy, gz, _ = gs[ch % 2]

            def ew(v, carry):
                vi = pl.ds(v * 16, 16)
                vj = pl.ds(BMAX + v * 16, 16)
                gx[vi] = gx[vj] - gx[vi]
                gy[vi] = gy[vj] - gy[vi]
                gz[vi] = gz[vj] - gz[vi]
                return carry

            lax.fori_loop(0, sz // 16, ew, 0, unroll=5)

        for d in in_descs(0):
            d.start()
        for d in in_descs(0):
            d.wait()
        for d in g_descs(0):
            d.start()
        for d in in_descs(1):
            d.start()

        for ch in range(nch):
            if ch + 1 < nch:
                for d in in_descs(ch + 1):
                    d.wait()
                if ch >= 1:
                    for d in out_descs(ch - 1):
                        d.wait()
                for d in g_descs(ch + 1):
                    d.start()
            for d in g_descs(ch):
                d.wait()
            if ch + 2 < nch:
                for d in in_descs(ch + 2):
                    d.start()
            compute(ch)
            for d in out_descs(ch):
                d.start()

        for d in out_descs(nch - 2):
            d.wait()
        for d in out_descs(nch - 1):
            d.wait()

    return _body


def _make_kernel(esub, part_base):
    return functools.partial(
        pl.kernel,
        mesh=plsc.VectorSubcoreMesh(core_axis_name="c", subcore_axis_name="s"),
        out_type=(
            jax.ShapeDtypeStruct((esub,), jnp.float32),
            jax.ShapeDtypeStruct((esub,), jnp.float32),
            jax.ShapeDtypeStruct((esub,), jnp.float32),
        ),
        compiler_params=pltpu.CompilerParams(
            needs_layout_passes=False, use_tc_tiling_on_sc=False),
        scratch_types=[
            pltpu.VMEM_SHARED((N,), jnp.float32),
            pltpu.VMEM_SHARED((N,), jnp.float32),
            pltpu.VMEM_SHARED((N,), jnp.float32),
            pltpu.VMEM((2 * BMAX,), jnp.int32),
            pltpu.VMEM((2 * BMAX,), jnp.int32),
            pltpu.VMEM((2 * BMAX,), jnp.float32),
            pltpu.VMEM((2 * BMAX,), jnp.float32),
            pltpu.VMEM((2 * BMAX,), jnp.float32),
            pltpu.VMEM((2 * BMAX,), jnp.float32),
            pltpu.VMEM((2 * BMAX,), jnp.float32),
            pltpu.VMEM((2 * BMAX,), jnp.float32),
            pltpu.SemaphoreType.DMA,
            pltpu.SemaphoreType.DMA,
            pltpu.SemaphoreType.DMA,
            pltpu.SemaphoreType.DMA,
            pltpu.SemaphoreType.DMA,
            pltpu.SemaphoreType.DMA,
        ],
    )(lambda *refs: _make_body(esub // NW, part_base)(*refs))


_part_bases = [sum(SPLITS[:k]) for k in range(len(SPLITS))]
_sc_parts = [_make_kernel(esub, base)
             for esub, base in zip(SPLITS, _part_bases)]


@jax.jit
def kernel(R, offsets, idx_i, idx_j):
    rx, ry, rz = R[:, 0], R[:, 1], R[:, 2]
    ii = idx_i.astype(jnp.int32)
    ij = idx_j.astype(jnp.int32)
    parts = []
    base = 0
    for esub, sc_part in zip(SPLITS, _sc_parts):
        dx, dy, dz = sc_part(rx, ry, rz, ii, ij)
        parts.append(jnp.stack([dx, dy, dz], axis=-1)
                     + offsets[base:base + esub])
        base += esub
    return jnp.concatenate(parts, axis=0)
